# Initial kernel scaffold; baseline (speedup 1.0000x reference)
import jax
import jax.numpy as jnp
from jax.experimental import pallas as pl


def kernel(x, edge_index, edge_type, emb, W_msg, b_msg, W_ih, W_hh, b_ih, b_hh):
    raise NotImplementedError("write your pallas kernel here")


# speedup vs baseline: 2.9047x; 2.9047x over previous
import functools

import jax
import jax.numpy as jnp
from jax import lax
from jax.experimental import pallas as pl
from jax.experimental.pallas import tpu as pltpu
from jax.experimental.pallas import tpu_sc as plsc

NC = 2
NS = 16
NW = NC * NS


def _sc_mesh():
    return plsc.VectorSubcoreMesh(
        core_axis_name="c", subcore_axis_name="s",
        num_cores=NC, num_subcores=NS)


def _emb_gather(emb, xp, n_pad, hd):
    nech, ech = xp.shape[1], xp.shape[2]
    per_w = nech * ech

    @functools.partial(
        pl.kernel,
        out_type=jax.ShapeDtypeStruct((n_pad, hd), jnp.float32),
        mesh=_sc_mesh(),
        scratch_types=[
            pltpu.VMEM((nech, ech), jnp.int32),
            pltpu.VMEM((ech, hd), jnp.float32),
            pltpu.SemaphoreType.DMA,
        ],
    )
    def k(emb_hbm, x_hbm, out_hbm, idx_v, rows_v, sem):
        c = lax.axis_index("c")
        s = lax.axis_index("s")
        wid = s * NC + c
        pltpu.sync_copy(x_hbm.at[wid], idx_v)

        def chunk(i, carry):
            off = pl.multiple_of(wid * per_w + i * ech, 8)
            pltpu.async_copy(emb_hbm.at[idx_v.at[i]], rows_v, sem).wait()
            pltpu.sync_copy(rows_v, out_hbm.at[pl.ds(off, ech)])
            return carry

        lax.fori_loop(0, nech, chunk, 0)

    return k(emb, xp)


def _sc_aggregate(allt_flat, gidx3, dst3, zeros, n2, hd):
    nch, ch = gidx3.shape[1], gidx3.shape[2]
    rpt = n2 // NS

    @functools.partial(
        pl.kernel,
        out_type=jax.ShapeDtypeStruct((NC, n2, hd), jnp.float32),
        mesh=_sc_mesh(),
        scratch_types=[
            pltpu.VMEM((nch, ch), jnp.int32),
            pltpu.VMEM((nch, ch), jnp.int32),
            pltpu.VMEM((ch, hd), jnp.float32),
            pltpu.VMEM_SHARED((n2, hd), jnp.float32),
            pltpu.SemaphoreType.DMA,
        ],
    )
    def k(allt_hbm, gidx_hbm, dst_hbm, z_hbm, out_hbm,
          gi_v, di_v, rows_v, aggr_sh, sem):
        c = lax.axis_index("c")
        s = lax.axis_index("s")
        wid = s * NC + c
        roff = pl.multiple_of(s * rpt, 8)
        pltpu.sync_copy(gidx_hbm.at[wid], gi_v)
        pltpu.sync_copy(dst_hbm.at[wid], di_v)
        pltpu.sync_copy(z_hbm.at[pl.ds(roff, rpt)],
                        aggr_sh.at[pl.ds(roff, rpt)])
        plsc.subcore_barrier()

        def chunk(i, carry):
            pltpu.async_copy(allt_hbm.at[gi_v.at[i]], rows_v, sem).wait()
            pltpu.sync_copy(rows_v, aggr_sh.at[di_v.at[i]], add=True)
            return carry

        lax.fori_loop(0, nch, chunk, 0)
        plsc.subcore_barrier()
        pltpu.sync_copy(aggr_sh.at[pl.ds(roff, rpt)],
                        out_hbm.at[c, pl.ds(roff, rpt)])

    return k(allt_flat, gidx3, dst3, zeros)


def _msg_transform(h, W_msg, b_msg3, bn):
    n, hd = h.shape
    t = W_msg.shape[0]
    nb = n // bn

    def body(h_ref, w_ref, b_ref, o_ref):
        o_ref[0] = (jnp.dot(h_ref[...], w_ref[0],
                            preferred_element_type=jnp.float32) + b_ref[0])

    return pl.pallas_call(
        body,
        grid=(nb, t),
        in_specs=[
            pl.BlockSpec((bn, hd), lambda i, j: (i, 0)),
            pl.BlockSpec((1, hd, hd), lambda i, j: (j, 0, 0)),
            pl.BlockSpec((1, 1, hd), lambda i, j: (j, 0, 0)),
        ],
        out_specs=pl.BlockSpec((1, bn, hd), lambda i, j: (j, i, 0)),
        out_shape=jax.ShapeDtypeStruct((t, n, hd), jnp.float32),
    )(h, W_msg, b_msg3)


def _gru(parts, h, WihT, WhhT, bih2, bhh2, bn):
    n, hd = h.shape
    nb = n // bn

    def body(p_ref, h_ref, wi_ref, wh_ref, bi_ref, bh_ref, o_ref):
        a = p_ref[0] + p_ref[1]
        hv = h_ref[...]
        gi = jnp.dot(a, wi_ref[...],
                     preferred_element_type=jnp.float32) + bi_ref[...]
        gh = jnp.dot(hv, wh_ref[...],
                     preferred_element_type=jnp.float32) + bh_ref[...]
        r = jax.nn.sigmoid(gi[:, :hd] + gh[:, :hd])
        z = jax.nn.sigmoid(gi[:, hd:2 * hd] + gh[:, hd:2 * hd])
        nn = jnp.tanh(gi[:, 2 * hd:] + r * gh[:, 2 * hd:])
        o_ref[...] = (1.0 - z) * nn + z * hv

    h3 = 3 * hd
    return pl.pallas_call(
        body,
        grid=(nb,),
        in_specs=[
            pl.BlockSpec((NC, bn, hd), lambda i: (0, i, 0)),
            pl.BlockSpec((bn, hd), lambda i: (i, 0)),
            pl.BlockSpec((hd, h3), lambda i: (0, 0)),
            pl.BlockSpec((hd, h3), lambda i: (0, 0)),
            pl.BlockSpec((1, h3), lambda i: (0, 0)),
            pl.BlockSpec((1, h3), lambda i: (0, 0)),
        ],
        out_specs=pl.BlockSpec((bn, hd), lambda i: (i, 0)),
        out_shape=jax.ShapeDtypeStruct((n, hd), jnp.float32),
    )(parts, h, WihT, WhhT, bih2, bhh2)


def kernel(x, edge_index, edge_type, emb, W_msg, b_msg, W_ih, W_hh, b_ih, b_hh):
    n = x.shape[0]
    e = edge_index.shape[1]
    t, hd, _ = W_msg.shape

    ech = 80
    per_w = -(-n // (NW * ech)) * ech
    n_pad = per_w * NW
    xp = jnp.pad(x, (0, n_pad - n)).reshape(NW, per_w // ech, ech)
    h = _emb_gather(emb, xp, n_pad, hd)[:n]

    epw = e // NW
    ch = 80
    nch = epw // ch
    src = edge_index[0]
    dst = edge_index[1]
    gidx3 = (edge_type * n + src).reshape(NW, nch, ch)
    dst3 = dst.reshape(NW, nch, ch)
    n2 = -(-n // (NS * 8)) * (NS * 8)
    zeros = jnp.zeros((n2, hd), jnp.float32)

    b_msg3 = b_msg.reshape(t, 1, hd)
    WihT = W_ih.T
    WhhT = W_hh.T
    bih2 = b_ih.reshape(1, 3 * hd)
    bhh2 = b_hh.reshape(1, 3 * hd)

    bn = 1000
    for _ in range(5):
        allt = _msg_transform(h, W_msg, b_msg3, bn)
        parts = _sc_aggregate(allt.reshape(t * n, hd), gidx3, dst3, zeros, n2, hd)
        h = _gru(parts, h, WihT, WhhT, bih2, bhh2, bn)
    return h
